# baseline (speedup 1.0000x reference)
import jax
import jax.numpy as jnp
from jax import lax
from jax.experimental import pallas as pl
from jax.experimental.pallas import tpu as pltpu
from jax.experimental.pallas import tpu_sc as plsc

S = 16384
Y = 20
D = 64
VOCAB = 100000
NW = 32
R = S // NW
G = 32
NCHUNK = R // G
PAIRS = G * Y
NSTEP = PAIRS // 128


def _sc_body(gram_h, ctx_h, wt_h, ct_h, wb_h, cb_h, out_h,
             widx0, wrows0, cidx0, crows0, wbv0, cbv0, dots0,
             widx1, wrows1, cidx1, crows1, wbv1, cbv1, dots1,
             isem0, isem1, dsem0, dsem1, osem0, osem1):
  wid = lax.axis_index("s") * 2 + lax.axis_index("c")
  wbase = wid * R

  lanes = lax.iota(jnp.int32, 16)
  lane15 = lanes == 15
  buf0 = (widx0, wrows0, cidx0, crows0, wbv0, cbv0, dots0,
          isem0, dsem0, osem0)
  buf1 = (widx1, wrows1, cidx1, crows1, wbv1, cbv1, dots1,
          isem1, dsem1, osem1)

  def pbase_of(ci):
    gbase = pl.multiple_of(wbase + ci * G, G)
    return pl.multiple_of(gbase * Y, 128)

  def fire_idx(ci, buf):
    widx_v, _, cidx_v, _, _, _, _, isem, _, _ = buf
    gbase = pl.multiple_of(wbase + ci * G, G)
    pbase = pbase_of(ci)
    pltpu.async_copy(gram_h.at[pl.ds(gbase, G)], widx_v, isem)
    for j in range(NSTEP):
      pltpu.async_copy(ctx_h.at[pl.ds(pbase + j * 128, 128)], cidx_v.at[j],
                       isem)

  def drain_idx(buf):
    widx_v, _, cidx_v, _, _, _, _, isem, _, _ = buf
    pltpu.make_async_copy(gram_h.at[pl.ds(0, G)], widx_v, isem).wait()
    for j in range(NSTEP):
      pltpu.make_async_copy(ctx_h.at[pl.ds(0, 128)], cidx_v.at[j],
                            isem).wait()

  def fire_data(buf):
    widx_v, wrows_v, cidx_v, crows_v, wb_v, cb_v, _, _, dsem, _ = buf
    pltpu.async_copy(wt_h.at[widx_v], wrows_v, dsem)
    pltpu.async_copy(wb_h.at[widx_v], wb_v, dsem)
    for j in range(NSTEP):
      pltpu.async_copy(ct_h.at[cidx_v.at[j]],
                       crows_v.at[pl.ds(j * 128, 128)], dsem)
      pltpu.async_copy(cb_h.at[cidx_v.at[j]],
                       cb_v.at[pl.ds(j * 128, 128)], dsem)

  def drain_data(buf):
    widx_v, wrows_v, cidx_v, crows_v, wb_v, cb_v, _, _, dsem, _ = buf
    pltpu.make_async_copy(wt_h.at[pl.ds(0, G)], wrows_v, dsem).wait()
    pltpu.make_async_copy(wb_h.at[pl.ds(0, G)], wb_v, dsem).wait()
    for j in range(NSTEP):
      pltpu.make_async_copy(ct_h.at[pl.ds(0, 128)],
                            crows_v.at[pl.ds(j * 128, 128)], dsem).wait()
      pltpu.make_async_copy(cb_h.at[pl.ds(0, 128)],
                            cb_v.at[pl.ds(j * 128, 128)], dsem).wait()

  def fire_out(ci, buf):
    dots_v, osem = buf[6], buf[9]
    pltpu.async_copy(dots_v, out_h.at[pl.ds(pbase_of(ci), PAIRS)], osem)

  def drain_out(buf):
    dots_v, osem = buf[6], buf[9]
    pltpu.make_async_copy(dots_v, out_h.at[pl.ds(0, PAIRS)], osem).wait()

  def compute(ci, buf):
    widx_v, wrows_v, cidx_v, crows_v, wb_v, cb_v, dots_v = buf[:7]
    pbase = pbase_of(ci)


    idx15 = jnp.full((16,), 15, jnp.int32)

    @plsc.parallel_loop(0, G // 4, unroll=2)
    def grp_body(grp):
      g0 = grp * 4
      res = jnp.zeros((16,), jnp.float32)
      for gg in range(4):
        g = g0 + gg
        w0 = wrows_v[g, pl.ds(0, 16)]
        w1 = wrows_v[g, pl.ds(16, 16)]
        w2 = wrows_v[g, pl.ds(32, 16)]
        w3 = wrows_v[g, pl.ds(48, 16)]
        for y in range(Y):
          q = gg * Y + y
          p = g * Y + y
          s0 = w0 * crows_v[p, pl.ds(0, 16)]
          s1 = w1 * crows_v[p, pl.ds(16, 16)]
          s2 = w2 * crows_v[p, pl.ds(32, 16)]
          s3 = w3 * crows_v[p, pl.ds(48, 16)]
          tot = jnp.sum((s0 + s1) + (s2 + s3))
          res = jnp.where(lanes == (q % 16), tot, res)
          if q % 16 == 15:
            k = q // 16
            off = grp * 80 + k * 16
            gidx = (jnp.full((16,), off, jnp.int32) + lanes) // Y
            wbx = plsc.load_gather(wb_v, [gidx])
            dots_v[pl.ds(off, 16)] = res + wbx + cb_v[pl.ds(off, 16)]
            res = jnp.zeros((16,), jnp.float32)

  fire_idx(0, buf0)
  drain_idx(buf0)
  fire_data(buf0)
  fire_idx(1, buf1)

  def step(i, buf, other, guard):
    drain_data(buf)

    @pl.when(guard)
    def _():
      fire_idx(i + 2, buf)

    @pl.when(i + 1 < NCHUNK)
    def _():
      drain_idx(other)
      fire_data(other)

    @pl.when(i >= 2)
    def _():
      drain_out(buf)

    compute(i, buf)
    fire_out(i, buf)

  def loop_body(ii, carry):
    a = 2 * ii
    step(a, buf0, buf1, ii < NCHUNK // 2 - 1)
    step(a + 1, buf1, buf0, ii < NCHUNK // 2 - 1)
    return carry

  lax.fori_loop(0, NCHUNK // 2, loop_body, 0, unroll=False)
  drain_out(buf0)
  drain_out(buf1)


@jax.jit
def _sc_call(gram_flat, ctx_flat, word_table, context_table, word_bias,
             context_bias):
  mesh = plsc.VectorSubcoreMesh(core_axis_name="c", subcore_axis_name="s")
  dbuf = [
      pltpu.VMEM((G,), jnp.int32),
      pltpu.VMEM((G, D), jnp.float32),
      pltpu.VMEM((NSTEP, 128), jnp.int32),
      pltpu.VMEM((PAIRS, D), jnp.float32),
      pltpu.VMEM((G,), jnp.float32),
      pltpu.VMEM((PAIRS,), jnp.float32),
      pltpu.VMEM((PAIRS,), jnp.float32),
  ]
  return pl.kernel(
      _sc_body,
      out_type=jax.ShapeDtypeStruct((S * Y,), jnp.float32),
      mesh=mesh,
      compiler_params=pltpu.CompilerParams(
          needs_layout_passes=False, use_tc_tiling_on_sc=False),
      scratch_types=dbuf + dbuf + [pltpu.SemaphoreType.DMA] * 6,
  )(gram_flat, ctx_flat, word_table, context_table, word_bias, context_bias)


def kernel(gram, context, word_table, context_table, word_bias, context_bias):
  gram_flat = gram.reshape(S).astype(jnp.int32)
  ctx_flat = context.reshape(S * Y).astype(jnp.int32)
  out_flat = _sc_call(gram_flat, ctx_flat, word_table, context_table,
                      word_bias, context_bias)
  return out_flat.reshape(S, Y)


# speedup vs baseline: 1.0074x; 1.0074x over previous
import jax
import jax.numpy as jnp
from jax import lax
from jax.experimental import pallas as pl
from jax.experimental.pallas import tpu as pltpu
from jax.experimental.pallas import tpu_sc as plsc

S = 16384
Y = 20
D = 64
VOCAB = 100000
NW = 32
R = S // NW
G = 32
NCHUNK = R // G
PAIRS = G * Y
NSTEP = PAIRS // 128


def _sc_body(gram_h, ctx_h, wt_h, ct_h, wb_h, cb_h, out_h,
             widx0, wrows0, cidx0, crows0, wbv0, cbv0, dots0,
             widx1, wrows1, cidx1, crows1, wbv1, cbv1, dots1,
             isem0, isem1, dsem0, dsem1, osem0, osem1):
  wid = lax.axis_index("s") * 2 + lax.axis_index("c")
  wbase = wid * R

  lanes = lax.iota(jnp.int32, 16)
  lane15 = lanes == 15
  buf0 = (widx0, wrows0, cidx0, crows0, wbv0, cbv0, dots0,
          isem0, dsem0, osem0)
  buf1 = (widx1, wrows1, cidx1, crows1, wbv1, cbv1, dots1,
          isem1, dsem1, osem1)

  def pbase_of(ci):
    gbase = pl.multiple_of(wbase + ci * G, G)
    return pl.multiple_of(gbase * Y, 128)

  def fire_idx(ci, buf):
    widx_v, _, cidx_v, _, _, _, _, isem, _, _ = buf
    gbase = pl.multiple_of(wbase + ci * G, G)
    pbase = pbase_of(ci)
    pltpu.async_copy(gram_h.at[pl.ds(gbase, G)], widx_v, isem)
    for j in range(NSTEP):
      pltpu.async_copy(ctx_h.at[pl.ds(pbase + j * 128, 128)], cidx_v.at[j],
                       isem)

  def drain_idx(buf):
    widx_v, _, cidx_v, _, _, _, _, isem, _, _ = buf
    pltpu.make_async_copy(gram_h.at[pl.ds(0, G)], widx_v, isem).wait()
    for j in range(NSTEP):
      pltpu.make_async_copy(ctx_h.at[pl.ds(0, 128)], cidx_v.at[j],
                            isem).wait()

  def fire_data(buf):
    widx_v, wrows_v, cidx_v, crows_v, wb_v, cb_v, _, _, dsem, _ = buf
    pltpu.async_copy(wt_h.at[widx_v], wrows_v, dsem)
    pltpu.async_copy(wb_h.at[widx_v], wb_v, dsem)
    for j in range(NSTEP):
      pltpu.async_copy(ct_h.at[cidx_v.at[j]],
                       crows_v.at[pl.ds(j * 128, 128)], dsem)
      pltpu.async_copy(cb_h.at[cidx_v.at[j]],
                       cb_v.at[pl.ds(j * 128, 128)], dsem)

  def drain_data(buf):
    widx_v, wrows_v, cidx_v, crows_v, wb_v, cb_v, _, _, dsem, _ = buf
    pltpu.make_async_copy(wt_h.at[pl.ds(0, G)], wrows_v, dsem).wait()
    pltpu.make_async_copy(wb_h.at[pl.ds(0, G)], wb_v, dsem).wait()
    for j in range(NSTEP):
      pltpu.make_async_copy(ct_h.at[pl.ds(0, 128)],
                            crows_v.at[pl.ds(j * 128, 128)], dsem).wait()
      pltpu.make_async_copy(cb_h.at[pl.ds(0, 128)],
                            cb_v.at[pl.ds(j * 128, 128)], dsem).wait()

  def fire_out(ci, buf):
    dots_v, osem = buf[6], buf[9]
    pltpu.async_copy(dots_v, out_h.at[pl.ds(pbase_of(ci), PAIRS)], osem)

  def drain_out(buf):
    dots_v, osem = buf[6], buf[9]
    pltpu.make_async_copy(dots_v, out_h.at[pl.ds(0, PAIRS)], osem).wait()

  def compute(ci, buf):
    widx_v, wrows_v, cidx_v, crows_v, wb_v, cb_v, dots_v = buf[:7]
    pbase = pbase_of(ci)


    idx15 = jnp.full((16,), 15, jnp.int32)

    @plsc.parallel_loop(0, G // 4, unroll=1)
    def grp_body(grp):
      g0 = grp * 4
      res = jnp.zeros((16,), jnp.float32)
      for gg in range(4):
        g = g0 + gg
        w0 = wrows_v[g, pl.ds(0, 16)]
        w1 = wrows_v[g, pl.ds(16, 16)]
        w2 = wrows_v[g, pl.ds(32, 16)]
        w3 = wrows_v[g, pl.ds(48, 16)]
        for y in range(Y):
          q = gg * Y + y
          p = g * Y + y
          acc = w0 * crows_v[p, pl.ds(0, 16)]
          acc = acc + w1 * crows_v[p, pl.ds(16, 16)]
          acc = acc + w2 * crows_v[p, pl.ds(32, 16)]
          acc = acc + w3 * crows_v[p, pl.ds(48, 16)]
          tot = jnp.sum(acc)
          res = jnp.where(lanes == (q % 16), tot, res)
          if q % 16 == 15:
            k = q // 16
            off = grp * 80 + k * 16
            gidx = (jnp.full((16,), off, jnp.int32) + lanes) // Y
            wbx = plsc.load_gather(wb_v, [gidx])
            dots_v[pl.ds(off, 16)] = res + wbx + cb_v[pl.ds(off, 16)]
            res = jnp.zeros((16,), jnp.float32)

  fire_idx(0, buf0)
  drain_idx(buf0)
  fire_data(buf0)
  fire_idx(1, buf1)

  def step(i, buf, other, guard):
    drain_data(buf)

    @pl.when(guard)
    def _():
      fire_idx(i + 2, buf)

    @pl.when(i + 1 < NCHUNK)
    def _():
      drain_idx(other)
      fire_data(other)

    @pl.when(i >= 2)
    def _():
      drain_out(buf)

    compute(i, buf)
    fire_out(i, buf)

  def loop_body(ii, carry):
    a = 2 * ii
    step(a, buf0, buf1, ii < NCHUNK // 2 - 1)
    step(a + 1, buf1, buf0, ii < NCHUNK // 2 - 1)
    return carry

  lax.fori_loop(0, NCHUNK // 2, loop_body, 0, unroll=False)
  drain_out(buf0)
  drain_out(buf1)


@jax.jit
def _sc_call(gram_flat, ctx_flat, word_table, context_table, word_bias,
             context_bias):
  mesh = plsc.VectorSubcoreMesh(core_axis_name="c", subcore_axis_name="s")
  dbuf = [
      pltpu.VMEM((G,), jnp.int32),
      pltpu.VMEM((G, D), jnp.float32),
      pltpu.VMEM((NSTEP, 128), jnp.int32),
      pltpu.VMEM((PAIRS, D), jnp.float32),
      pltpu.VMEM((G,), jnp.float32),
      pltpu.VMEM((PAIRS,), jnp.float32),
      pltpu.VMEM((PAIRS,), jnp.float32),
  ]
  return pl.kernel(
      _sc_body,
      out_type=jax.ShapeDtypeStruct((S * Y,), jnp.float32),
      mesh=mesh,
      compiler_params=pltpu.CompilerParams(
          needs_layout_passes=False, use_tc_tiling_on_sc=False),
      scratch_types=dbuf + dbuf + [pltpu.SemaphoreType.DMA] * 6,
  )(gram_flat, ctx_flat, word_table, context_table, word_bias, context_bias)


def kernel(gram, context, word_table, context_table, word_bias, context_bias):
  gram_flat = gram.reshape(S).astype(jnp.int32)
  ctx_flat = context.reshape(S * Y).astype(jnp.int32)
  out_flat = _sc_call(gram_flat, ctx_flat, word_table, context_table,
                      word_bias, context_bias)
  return out_flat.reshape(S, Y)


# speedup vs baseline: 1.0094x; 1.0019x over previous
import jax
import jax.numpy as jnp
from jax import lax
from jax.experimental import pallas as pl
from jax.experimental.pallas import tpu as pltpu
from jax.experimental.pallas import tpu_sc as plsc

S = 16384
Y = 20
D = 64
VOCAB = 100000
NW = 32
R = S // NW
G = 32
NCHUNK = R // G
PAIRS = G * Y
NSTEP = PAIRS // 128


def _sc_body(gram_h, ctx_h, wt_h, ct_h, wb_h, cb_h, out_h,
             widx0, wrows0, cidx0, crows0, wbv0, cbv0, dots0,
             widx1, wrows1, cidx1, crows1, wbv1, cbv1, dots1,
             isem0, isem1, dsem0, dsem1, osem0, osem1):
  wid = lax.axis_index("s") * 2 + lax.axis_index("c")
  wbase = wid * R

  lanes = lax.iota(jnp.int32, 16)
  buf0 = (widx0, wrows0, cidx0, crows0, wbv0, cbv0, dots0,
          isem0, dsem0, osem0)
  buf1 = (widx1, wrows1, cidx1, crows1, wbv1, cbv1, dots1,
          isem1, dsem1, osem1)

  def pbase_of(ci):
    gbase = pl.multiple_of(wbase + ci * G, G)
    return pl.multiple_of(gbase * Y, 128)

  def fire_idx(ci, buf):
    widx_v, _, cidx_v, _, _, _, _, isem, _, _ = buf
    gbase = pl.multiple_of(wbase + ci * G, G)
    pbase = pbase_of(ci)
    pltpu.async_copy(gram_h.at[pl.ds(gbase, G)], widx_v, isem)
    for j in range(NSTEP):
      pltpu.async_copy(ctx_h.at[pl.ds(pbase + j * 128, 128)], cidx_v.at[j],
                       isem)

  def drain_idx(buf):
    widx_v, _, cidx_v, _, _, _, _, isem, _, _ = buf
    pltpu.make_async_copy(gram_h.at[pl.ds(0, G)], widx_v, isem).wait()
    for j in range(NSTEP):
      pltpu.make_async_copy(ctx_h.at[pl.ds(0, 128)], cidx_v.at[j],
                            isem).wait()

  def fire_data(buf):
    widx_v, wrows_v, cidx_v, crows_v, wb_v, cb_v, _, _, dsem, _ = buf
    pltpu.async_copy(wt_h.at[widx_v], wrows_v, dsem)
    pltpu.async_copy(wb_h.at[widx_v], wb_v, dsem)
    for j in range(NSTEP):
      pltpu.async_copy(ct_h.at[cidx_v.at[j]],
                       crows_v.at[pl.ds(j * 128, 128)], dsem)
      pltpu.async_copy(cb_h.at[cidx_v.at[j]],
                       cb_v.at[pl.ds(j * 128, 128)], dsem)

  def drain_data(buf):
    widx_v, wrows_v, cidx_v, crows_v, wb_v, cb_v, _, _, dsem, _ = buf
    pltpu.make_async_copy(wt_h.at[pl.ds(0, G)], wrows_v, dsem).wait()
    pltpu.make_async_copy(wb_h.at[pl.ds(0, G)], wb_v, dsem).wait()
    for j in range(NSTEP):
      pltpu.make_async_copy(ct_h.at[pl.ds(0, 128)],
                            crows_v.at[pl.ds(j * 128, 128)], dsem).wait()
      pltpu.make_async_copy(cb_h.at[pl.ds(0, 128)],
                            cb_v.at[pl.ds(j * 128, 128)], dsem).wait()

  def fire_out(ci, buf):
    dots_v, osem = buf[6], buf[9]
    pltpu.async_copy(dots_v, out_h.at[pl.ds(pbase_of(ci), PAIRS)], osem)

  def drain_out(buf):
    dots_v, osem = buf[6], buf[9]
    pltpu.make_async_copy(dots_v, out_h.at[pl.ds(0, PAIRS)], osem).wait()

  def compute(ci, buf):
    widx_v, wrows_v, cidx_v, crows_v, wb_v, cb_v, dots_v = buf[:7]

    @plsc.parallel_loop(0, G // 4, unroll=1)
    def grp_body(grp):
      g0 = grp * 4
      res = jnp.zeros((16,), jnp.float32)
      for gg in range(4):
        g = g0 + gg
        w0 = wrows_v[g, pl.ds(0, 16)]
        w1 = wrows_v[g, pl.ds(16, 16)]
        w2 = wrows_v[g, pl.ds(32, 16)]
        w3 = wrows_v[g, pl.ds(48, 16)]
        for y in range(Y):
          q = gg * Y + y
          p = g * Y + y
          s0 = w0 * crows_v[p, pl.ds(0, 16)]
          s1 = w1 * crows_v[p, pl.ds(16, 16)]
          s2 = w2 * crows_v[p, pl.ds(32, 16)]
          s3 = w3 * crows_v[p, pl.ds(48, 16)]
          tot = jnp.sum((s0 + s1) + (s2 + s3))
          res = jnp.where(lanes == (q % 16), tot, res)
          if q % 16 == 15:
            k = q // 16
            off = grp * 80 + k * 16
            gidx = (jnp.full((16,), off, jnp.int32) + lanes) // Y
            wbx = plsc.load_gather(wb_v, [gidx])
            dots_v[pl.ds(off, 16)] = res + wbx + cb_v[pl.ds(off, 16)]
            res = jnp.zeros((16,), jnp.float32)

  fire_idx(0, buf0)
  drain_idx(buf0)
  fire_data(buf0)
  fire_idx(1, buf1)

  def step(i, buf, other, guard):
    drain_data(buf)

    @pl.when(guard)
    def _():
      fire_idx(i + 2, buf)

    @pl.when(i + 1 < NCHUNK)
    def _():
      drain_idx(other)
      fire_data(other)

    @pl.when(i >= 2)
    def _():
      drain_out(buf)

    compute(i, buf)
    fire_out(i, buf)

  def loop_body(ii, carry):
    a = 2 * ii
    step(a, buf0, buf1, ii < NCHUNK // 2 - 1)
    step(a + 1, buf1, buf0, ii < NCHUNK // 2 - 1)
    return carry

  lax.fori_loop(0, NCHUNK // 2, loop_body, 0, unroll=False)
  drain_out(buf0)
  drain_out(buf1)


@jax.jit
def _sc_call(gram_flat, ctx_flat, word_table, context_table, word_bias,
             context_bias):
  mesh = plsc.VectorSubcoreMesh(core_axis_name="c", subcore_axis_name="s")
  dbuf = [
      pltpu.VMEM((G,), jnp.int32),
      pltpu.VMEM((G, D), jnp.float32),
      pltpu.VMEM((NSTEP, 128), jnp.int32),
      pltpu.VMEM((PAIRS, D), jnp.float32),
      pltpu.VMEM((G,), jnp.float32),
      pltpu.VMEM((PAIRS,), jnp.float32),
      pltpu.VMEM((PAIRS,), jnp.float32),
  ]
  return pl.kernel(
      _sc_body,
      out_type=jax.ShapeDtypeStruct((S * Y,), jnp.float32),
      mesh=mesh,
      compiler_params=pltpu.CompilerParams(
          needs_layout_passes=False, use_tc_tiling_on_sc=False),
      scratch_types=dbuf + dbuf + [pltpu.SemaphoreType.DMA] * 6,
  )(gram_flat, ctx_flat, word_table, context_table, word_bias, context_bias)


def kernel(gram, context, word_table, context_table, word_bias, context_bias):
  gram_flat = gram.reshape(S).astype(jnp.int32)
  ctx_flat = context.reshape(S * Y).astype(jnp.int32)
  out_flat = _sc_call(gram_flat, ctx_flat, word_table, context_table,
                      word_bias, context_bias)
  return out_flat.reshape(S, Y)
